# Initial kernel scaffold; baseline (speedup 1.0000x reference)
import jax
import jax.numpy as jnp
from jax.experimental import pallas as pl


def kernel(xyz1, xyz2, points1, points2, W1, b1, g1, be1, W2, b2, g2, be2):
    raise NotImplementedError("write your pallas kernel here")


# speedup vs baseline: 22.9807x; 22.9807x over previous
import functools

import jax
import jax.numpy as jnp
from jax.experimental import pallas as pl

B, N, S, D1, D2 = 8, 4096, 1024, 128, 256
C1, C2 = 256, 128
BN_ROWS = 512
NB = N // BN_ROWS
M = B * N


def _k1_body(x1_ref, x2t_ref, p2_ref, p1_ref, w1a_ref, w1b_ref, b1_ref,
             h1_ref, ssum_ref, ssq_ref):
    x1 = x1_ref[...]
    x2t = x2t_ref[...]
    sq1 = jnp.sum(x1 * x1, axis=1, keepdims=True)
    sq2 = jnp.sum(x2t * x2t, axis=0, keepdims=True)
    xx = jax.lax.dot_general(x1, x2t, (((1,), (0,)), ((), ())),
                             preferred_element_type=jnp.float32)
    dist = jnp.maximum(sq1 + sq2 - 2.0 * xx, 0.0)

    iota = jax.lax.broadcasted_iota(jnp.int32, (BN_ROWS, S), 1)
    d = dist
    vals = []
    idxs = []
    for _ in range(3):
        mk = jnp.min(d, axis=1, keepdims=True)
        ik = jnp.min(jnp.where(d == mk, iota, S), axis=1, keepdims=True)
        vals.append(mk)
        idxs.append(ik)
        d = jnp.where(iota == ik, jnp.inf, d)

    r = [1.0 / (v + 1e-8) for v in vals]
    norm = r[0] + r[1] + r[2]
    w = [ri / norm for ri in r]

    wd = (jnp.where(iota == idxs[0], w[0], 0.0)
          + jnp.where(iota == idxs[1], w[1], 0.0)
          + jnp.where(iota == idxs[2], w[2], 0.0))
    interp = jax.lax.dot_general(wd, p2_ref[...], (((1,), (0,)), ((), ())),
                                 preferred_element_type=jnp.float32)

    h1 = (jax.lax.dot_general(p1_ref[...], w1a_ref[...],
                              (((1,), (0,)), ((), ())),
                              preferred_element_type=jnp.float32)
          + jax.lax.dot_general(interp, w1b_ref[...],
                                (((1,), (0,)), ((), ())),
                                preferred_element_type=jnp.float32)
          + b1_ref[...])
    h1_ref[...] = h1

    first = (pl.program_id(0) == 0) & (pl.program_id(1) == 0)

    @pl.when(first)
    def _():
        ssum_ref[...] = jnp.zeros_like(ssum_ref)
        ssq_ref[...] = jnp.zeros_like(ssq_ref)

    ssum_ref[...] += jnp.sum(h1, axis=0, keepdims=True)
    ssq_ref[...] += jnp.sum(h1 * h1, axis=0, keepdims=True)


def _k2_body(h1_ref, sc_ref, sh_ref, w2t_ref, b2_ref,
             h2_ref, ssum_ref, ssq_ref):
    h1n = jnp.maximum(h1_ref[...] * sc_ref[...] + sh_ref[...], 0.0)
    h2 = (jax.lax.dot_general(h1n, w2t_ref[...], (((1,), (0,)), ((), ())),
                              preferred_element_type=jnp.float32)
          + b2_ref[...])
    h2_ref[...] = h2

    @pl.when(pl.program_id(0) == 0)
    def _():
        ssum_ref[...] = jnp.zeros_like(ssum_ref)
        ssq_ref[...] = jnp.zeros_like(ssq_ref)

    ssum_ref[...] += jnp.sum(h2, axis=0, keepdims=True)
    ssq_ref[...] += jnp.sum(h2 * h2, axis=0, keepdims=True)


def _k3_body(h2_ref, sc_ref, sh_ref, out_ref):
    out_ref[...] = jnp.maximum(h2_ref[...] * sc_ref[...] + sh_ref[...], 0.0)


def _affine(ssum, ssq, gamma, beta):
    mean = ssum[0] / M
    var = ssq[0] / M - mean * mean
    scale = gamma * jax.lax.rsqrt(var + 1e-5)
    shift = beta - mean * scale
    return scale[None, :], shift[None, :]


@jax.jit
def kernel(xyz1, xyz2, points1, points2, W1, b1, g1, be1, W2, b2, g2, be2):
    x1f = xyz1.reshape(M, 3)
    x2t = jnp.transpose(xyz2, (0, 2, 1))
    p1f = points1.reshape(M, D1)
    w1a = W1[:, :D1].T
    w1b = W1[:, D1:].T
    w2t = W2.T

    rowblk = lambda r, c: pl.BlockSpec((r, c), lambda b, n: (b * NB + n, 0))
    perb = lambda d0, d1: pl.BlockSpec((None, d0, d1), lambda b, n: (b, 0, 0))
    full = lambda d0, d1: pl.BlockSpec((d0, d1), lambda b, n: (0, 0))

    h1, s1, q1 = pl.pallas_call(
        _k1_body,
        grid=(B, NB),
        in_specs=[rowblk(BN_ROWS, 3), perb(3, S), perb(S, D2),
                  rowblk(BN_ROWS, D1), full(D1, C1), full(D2, C1),
                  full(1, C1)],
        out_specs=[rowblk(BN_ROWS, C1), full(1, C1), full(1, C1)],
        out_shape=[jax.ShapeDtypeStruct((M, C1), jnp.float32),
                   jax.ShapeDtypeStruct((1, C1), jnp.float32),
                   jax.ShapeDtypeStruct((1, C1), jnp.float32)],
    )(x1f, x2t, points2, p1f, w1a, w1b, b1[None, :])

    sc1, sh1 = _affine(s1, q1, g1, be1)

    blk = lambda r, c: pl.BlockSpec((r, c), lambda i: (i, 0))
    full1 = lambda d0, d1: pl.BlockSpec((d0, d1), lambda i: (0, 0))

    h2, s2, q2 = pl.pallas_call(
        _k2_body,
        grid=(M // BN_ROWS,),
        in_specs=[blk(BN_ROWS, C1), full1(1, C1), full1(1, C1),
                  full1(C1, C2), full1(1, C2)],
        out_specs=[blk(BN_ROWS, C2), full1(1, C2), full1(1, C2)],
        out_shape=[jax.ShapeDtypeStruct((M, C2), jnp.float32),
                   jax.ShapeDtypeStruct((1, C2), jnp.float32),
                   jax.ShapeDtypeStruct((1, C2), jnp.float32)],
    )(h1, sc1, sh1, w2t, b2[None, :])

    sc2, sh2 = _affine(s2, q2, g2, be2)

    out = pl.pallas_call(
        _k3_body,
        grid=(M // BN_ROWS,),
        in_specs=[blk(BN_ROWS, C2), full1(1, C2), full1(1, C2)],
        out_specs=blk(BN_ROWS, C2),
        out_shape=jax.ShapeDtypeStruct((M, C2), jnp.float32),
    )(h2, sc2, sh2)

    return out.reshape(B, N, C2)
